# Initial kernel scaffold; baseline (speedup 1.0000x reference)
import jax
import jax.numpy as jnp
from jax.experimental import pallas as pl


def kernel(word_id, tag_id, predicate, word_table, tag_table):
    raise NotImplementedError("write your pallas kernel here")


# speedup vs baseline: 1.1058x; 1.1058x over previous
import functools

import jax
import jax.numpy as jnp
from jax import lax
from jax.experimental import pallas as pl
from jax.experimental.pallas import tpu as pltpu
from jax.experimental.pallas import tpu_sc as plsc

WORD_DIM = 128
TAG_DIM = 32
PRED_SIZE = 16
OUT_DIM = WORD_DIM + TAG_DIM + PRED_SIZE

NUM_CORES = 2
NUM_SUBCORES = 16
NUM_WORKERS = NUM_CORES * NUM_SUBCORES
CHUNK = 128


@functools.partial(jax.jit, static_argnames=("rows",))
def _sc_embed(word_id, tag_id, predicate, word_table, tag_table, pred_table,
              rows: int):
    rows_per_w = rows // NUM_WORKERS
    chunks = rows_per_w // CHUNK
    mesh = plsc.VectorSubcoreMesh(core_axis_name="c", subcore_axis_name="s")

    @functools.partial(
        pl.kernel,
        out_type=jax.ShapeDtypeStruct((rows, OUT_DIM), jnp.float32),
        mesh=mesh,
        compiler_params=pltpu.CompilerParams(use_tc_tiling_on_sc=False),
        scratch_types=[
            pltpu.VMEM((CHUNK,), jnp.int32),
            pltpu.VMEM((CHUNK,), jnp.int32),
            pltpu.VMEM((CHUNK,), jnp.int32),
            pltpu.VMEM((CHUNK, WORD_DIM), jnp.float32),
            pltpu.VMEM((CHUNK, TAG_DIM), jnp.float32),
            pltpu.VMEM((CHUNK, PRED_SIZE), jnp.float32),
            pltpu.SemaphoreType.DMA,
            pltpu.SemaphoreType.DMA,
            pltpu.SemaphoreType.DMA,
        ],
    )
    def k(wid_hbm, tid_hbm, pid_hbm, wtab_hbm, ttab_hbm, ptab_hbm, out_hbm,
          widx_v, tidx_v, pidx_v, wrows_v, trows_v, prows_v,
          sem_w, sem_t, sem_p):
        w = lax.axis_index("s") * NUM_CORES + lax.axis_index("c")
        wbase = w * rows_per_w

        def body(c, carry):
            base = wbase + c * CHUNK
            pltpu.sync_copy(wid_hbm.at[pl.ds(base, CHUNK)], widx_v)
            pltpu.sync_copy(tid_hbm.at[pl.ds(base, CHUNK)], tidx_v)
            pltpu.sync_copy(pid_hbm.at[pl.ds(base, CHUNK)], pidx_v)
            cw = pltpu.async_copy(wtab_hbm.at[widx_v], wrows_v, sem_w)
            ct = pltpu.async_copy(ttab_hbm.at[tidx_v], trows_v, sem_t)
            cp = pltpu.async_copy(ptab_hbm.at[pidx_v], prows_v, sem_p)
            cw.wait()
            ct.wait()
            cp.wait()
            pltpu.sync_copy(wrows_v,
                            out_hbm.at[pl.ds(base, CHUNK), pl.ds(0, WORD_DIM)])
            pltpu.sync_copy(trows_v,
                            out_hbm.at[pl.ds(base, CHUNK),
                                       pl.ds(WORD_DIM, TAG_DIM)])
            pltpu.sync_copy(prows_v,
                            out_hbm.at[pl.ds(base, CHUNK),
                                       pl.ds(WORD_DIM + TAG_DIM, PRED_SIZE)])
            return carry

        lax.fori_loop(0, chunks, body, 0)

    return k(word_id, tag_id, predicate, word_table, tag_table, pred_table)


def kernel(word_id, tag_id, predicate, word_table, tag_table):
    B, L = word_id.shape
    rows = B * L
    pred_table = jnp.concatenate(
        [jnp.zeros((1, PRED_SIZE), jnp.float32),
         jnp.ones((1, PRED_SIZE), jnp.float32)], axis=0)
    out = _sc_embed(word_id.reshape(rows), tag_id.reshape(rows),
                    predicate.reshape(rows), word_table, tag_table,
                    pred_table, rows=rows)
    return out.reshape(B, L, OUT_DIM)
